# baseline (speedup 1.0000x reference)
import jax
import jax.numpy as jnp
from jax import lax
from jax.experimental import pallas as pl
from jax.experimental.pallas import tpu as pltpu

_TOKENS_PER_TILE = 2048

_DN_RHS_T = (((1,), (1,)), ((), ()))


def _moe_lora_tile(x_ref, wr_ref, a_ref, b_ref, alpha_ref, out_ref,
                   wrs_ref, a2s_ref, b2s_ref):
    k, r, d = a_ref.shape
    kr = k * r

    @pl.when(pl.program_id(0) == 0)
    def _prep():
        wrs_ref[...] = jnp.repeat(wr_ref[...], r, axis=0)
        a2s_ref[...] = a_ref[...].reshape(kr, d).astype(jnp.bfloat16)
        b2s_ref[...] = (jnp.transpose(b_ref[...], (0, 2, 1)).reshape(kr, d)
                        * alpha_ref[0]).astype(jnp.bfloat16)

    x = x_ref[...]
    scores = lax.dot_general(x, wrs_ref[...], _DN_RHS_T,
                             preferred_element_type=jnp.float32)
    m1 = jnp.max(scores, axis=1, keepdims=True)
    is1 = scores == m1
    s2 = jnp.where(is1, -jnp.inf, scores)
    m2 = jnp.max(s2, axis=1, keepdims=True)
    g1 = 1.0 / (1.0 + jnp.exp(m2 - m1))
    g2 = 1.0 - g1
    w = jnp.where(is1, g1, 0.0) + jnp.where(s2 == m2, g2, 0.0)

    h = lax.dot_general(x.astype(jnp.bfloat16), a2s_ref[...], _DN_RHS_T,
                        preferred_element_type=jnp.float32)
    out_ref[...] = jnp.dot((h * w).astype(jnp.bfloat16), b2s_ref[...],
                           preferred_element_type=jnp.float32)


def kernel(x, A, Bmat, Wr, alpha_over_r):
    b, s, d = x.shape
    k, r, _ = A.shape
    kr = k * r
    n_tok = b * s
    tile = _TOKENS_PER_TILE

    x2 = x.reshape(n_tok, d)
    alpha = jnp.asarray(alpha_over_r, jnp.float32).reshape(1)

    out = pl.pallas_call(
        _moe_lora_tile,
        grid=(n_tok // tile,),
        in_specs=[
            pl.BlockSpec((tile, d), lambda i: (i, 0)),
            pl.BlockSpec((k, d), lambda i: (0, 0)),
            pl.BlockSpec((k, r, d), lambda i: (0, 0, 0)),
            pl.BlockSpec((k, d, r), lambda i: (0, 0, 0)),
            pl.BlockSpec(memory_space=pltpu.SMEM),
        ],
        out_specs=pl.BlockSpec((tile, d), lambda i: (i, 0)),
        out_shape=jax.ShapeDtypeStruct((n_tok, d), x.dtype),
        scratch_shapes=[
            pltpu.VMEM((kr, d), jnp.float32),
            pltpu.VMEM((kr, d), jnp.bfloat16),
            pltpu.VMEM((kr, d), jnp.bfloat16),
        ],
        compiler_params=pltpu.CompilerParams(
            dimension_semantics=("arbitrary",)),
    )(x2, Wr, A, Bmat, alpha)
    return out.reshape(b, s, d)


# speedup vs baseline: 1.0078x; 1.0078x over previous
import jax
import jax.numpy as jnp
from jax import lax
from jax.experimental import pallas as pl
from jax.experimental.pallas import tpu as pltpu

_TILE = 1024

_DN_RHS_T = (((1,), (1,)), ((), ()))


def _compute_tile(x, wr_rep, a2, b2):
    scores = lax.dot_general(x, wr_rep, _DN_RHS_T,
                             preferred_element_type=jnp.float32)
    m1 = jnp.max(scores, axis=1, keepdims=True)
    is1 = scores == m1
    s2 = jnp.where(is1, -jnp.inf, scores)
    m2 = jnp.max(s2, axis=1, keepdims=True)
    g1 = 1.0 / (1.0 + jnp.exp(m2 - m1))
    g2 = 1.0 - g1
    w = jnp.where(is1, g1, 0.0) + jnp.where(s2 == m2, g2, 0.0)

    h = lax.dot_general(x.astype(jnp.bfloat16), a2, _DN_RHS_T,
                        preferred_element_type=jnp.float32)
    return jnp.dot((h * w).astype(jnp.bfloat16), b2,
                   preferred_element_type=jnp.float32)


def _moe_lora_kernel(x_hbm, wr_ref, a_ref, b2_ref, out_hbm,
                     xbuf, obuf, in_sem, out_sem):
    n = x_hbm.shape[0]
    nsteps = n // _TILE

    def in_copy(i, slot):
        return pltpu.make_async_copy(
            x_hbm.at[pl.ds(i * _TILE, _TILE), :], xbuf.at[slot],
            in_sem.at[slot])

    def out_copy(i, slot):
        return pltpu.make_async_copy(
            obuf.at[slot], out_hbm.at[pl.ds(i * _TILE, _TILE), :],
            out_sem.at[slot])

    in_copy(0, 0).start()
    in_copy(1, 1).start()

    kr = a_ref.shape[0] * a_ref.shape[1]
    d = a_ref.shape[2]
    a2 = a_ref[...].reshape(kr, d).astype(jnp.bfloat16)
    wr_rep = wr_ref[...]
    b2 = b2_ref[...]

    def step(i, carry):
        slot = lax.rem(i, 2)
        in_copy(i, slot).wait()
        res = _compute_tile(xbuf[slot], wr_rep, a2, b2)

        @pl.when(i >= 2)
        def _():
            out_copy(i - 2, slot).wait()

        obuf[slot] = res
        out_copy(i, slot).start()

        @pl.when(i + 2 < nsteps)
        def _():
            in_copy(i + 2, slot).start()

        return carry

    lax.fori_loop(0, nsteps, step, 0)
    out_copy(nsteps - 2, (nsteps - 2) % 2).wait()
    out_copy(nsteps - 1, (nsteps - 1) % 2).wait()


def kernel(x, A, Bmat, Wr, alpha_over_r):
    b, s, d = x.shape
    k, r, _ = A.shape
    kr = k * r
    n_tok = b * s

    x2 = x.reshape(n_tok, d)
    wr_rep = jnp.repeat(Wr, r, axis=0)
    b2 = (Bmat.transpose(0, 2, 1).reshape(kr, d)
          * jnp.asarray(alpha_over_r, x.dtype)).astype(jnp.bfloat16)

    out = pl.pallas_call(
        _moe_lora_kernel,
        in_specs=[
            pl.BlockSpec(memory_space=pltpu.HBM),
            pl.BlockSpec(memory_space=pltpu.VMEM),
            pl.BlockSpec(memory_space=pltpu.VMEM),
            pl.BlockSpec(memory_space=pltpu.VMEM),
        ],
        out_specs=pl.BlockSpec(memory_space=pltpu.HBM),
        out_shape=jax.ShapeDtypeStruct((n_tok, d), x.dtype),
        scratch_shapes=[
            pltpu.VMEM((2, _TILE, d), jnp.float32),
            pltpu.VMEM((2, _TILE, d), jnp.float32),
            pltpu.SemaphoreType.DMA((2,)),
            pltpu.SemaphoreType.DMA((2,)),
        ],
    )(x2, wr_rep, A, b2)
    return out.reshape(b, s, d)


# speedup vs baseline: 1.0300x; 1.0221x over previous
import jax
import jax.numpy as jnp
from jax import lax
from jax.experimental import pallas as pl
from jax.experimental.pallas import tpu as pltpu

_TILE = 2048

_DN_RHS_T = (((1,), (1,)), ((), ()))


def _compute_tile(x, wr_rep, a2, b2):
    scores = lax.dot_general(x, wr_rep, _DN_RHS_T,
                             preferred_element_type=jnp.float32)
    m1 = jnp.max(scores, axis=1, keepdims=True)
    is1 = scores == m1
    s2 = jnp.where(is1, -jnp.inf, scores)
    m2 = jnp.max(s2, axis=1, keepdims=True)
    g1 = 1.0 / (1.0 + jnp.exp(m2 - m1))
    g2 = 1.0 - g1
    w = jnp.where(is1, g1, 0.0) + jnp.where(s2 == m2, g2, 0.0)

    h = lax.dot_general(x.astype(jnp.bfloat16), a2, _DN_RHS_T,
                        preferred_element_type=jnp.float32)
    return jnp.dot((h * w).astype(jnp.bfloat16), b2,
                   preferred_element_type=jnp.float32)


def _moe_lora_kernel(x_hbm, wr_ref, a_ref, b2_ref, out_hbm,
                     xbuf, obuf, in_sem, out_sem):
    n = x_hbm.shape[0]
    nsteps = n // _TILE

    def in_copy(i, slot):
        return pltpu.make_async_copy(
            x_hbm.at[pl.ds(i * _TILE, _TILE), :], xbuf.at[slot],
            in_sem.at[slot])

    def out_copy(i, slot):
        return pltpu.make_async_copy(
            obuf.at[slot], out_hbm.at[pl.ds(i * _TILE, _TILE), :],
            out_sem.at[slot])

    in_copy(0, 0).start()
    in_copy(1, 1).start()

    kr = a_ref.shape[0] * a_ref.shape[1]
    d = a_ref.shape[2]
    a2 = a_ref[...].reshape(kr, d).astype(jnp.bfloat16)
    wr_rep = wr_ref[...]
    b2 = b2_ref[...]

    def step(i, carry):
        slot = lax.rem(i, 2)
        in_copy(i, slot).wait()
        res = _compute_tile(xbuf[slot], wr_rep, a2, b2)

        @pl.when(i >= 2)
        def _():
            out_copy(i - 2, slot).wait()

        obuf[slot] = res
        out_copy(i, slot).start()

        @pl.when(i + 2 < nsteps)
        def _():
            in_copy(i + 2, slot).start()

        return carry

    lax.fori_loop(0, nsteps, step, 0)
    out_copy(nsteps - 2, (nsteps - 2) % 2).wait()
    out_copy(nsteps - 1, (nsteps - 1) % 2).wait()


def kernel(x, A, Bmat, Wr, alpha_over_r):
    b, s, d = x.shape
    k, r, _ = A.shape
    kr = k * r
    n_tok = b * s

    x2 = x.reshape(n_tok, d)
    wr_rep = jnp.repeat(Wr, r, axis=0)
    b2 = (Bmat.transpose(0, 2, 1).reshape(kr, d)
          * jnp.asarray(alpha_over_r, x.dtype)).astype(jnp.bfloat16)

    out = pl.pallas_call(
        _moe_lora_kernel,
        in_specs=[
            pl.BlockSpec(memory_space=pltpu.HBM),
            pl.BlockSpec(memory_space=pltpu.VMEM),
            pl.BlockSpec(memory_space=pltpu.VMEM),
            pl.BlockSpec(memory_space=pltpu.VMEM),
        ],
        out_specs=pl.BlockSpec(memory_space=pltpu.HBM),
        out_shape=jax.ShapeDtypeStruct((n_tok, d), x.dtype),
        scratch_shapes=[
            pltpu.VMEM((2, _TILE, d), jnp.float32),
            pltpu.VMEM((2, _TILE, d), jnp.float32),
            pltpu.SemaphoreType.DMA((2,)),
            pltpu.SemaphoreType.DMA((2,)),
        ],
    )(x2, wr_rep, A, b2)
    return out.reshape(b, s, d)
